# baseline (speedup 1.0000x reference)
import functools

import jax
import jax.numpy as jnp
from jax import lax
from jax.experimental import pallas as pl
from jax.experimental.pallas import tpu as pltpu
from jax.experimental.pallas import tpu_sc as plsc

D = 64
F = 128

NC = 2
NS = 16
NW = NC * NS

IDX_ROW = 128
SUBS = 4
CHUNK = IDX_ROW * SUBS


def _gather_body(idx_hbm, table_hbm, out_hbm, idx_v, rows_v, sem, *, n_chunks):
    wid = lax.axis_index("s") * NC + lax.axis_index("c")
    chunk_rows = SUBS
    base_row = wid * (n_chunks * chunk_rows)

    def body(c, carry):
        row_off = base_row + c * chunk_rows
        tok_off = row_off * IDX_ROW
        pltpu.sync_copy(idx_hbm.at[pl.ds(row_off, chunk_rows)], idx_v)
        copies = []
        for j in range(SUBS):
            copies.append(
                pltpu.async_copy(
                    table_hbm.at[idx_v.at[j]],
                    rows_v.at[pl.ds(j * IDX_ROW, IDX_ROW)],
                    sem,
                )
            )
        for cp in copies:
            cp.wait()
        pltpu.sync_copy(rows_v, out_hbm.at[pl.ds(tok_off, CHUNK)])
        return carry

    lax.fori_loop(0, n_chunks, body, 0)


@functools.lru_cache(maxsize=None)
def _make_gather(tok):
    n_chunks = tok // (NW * CHUNK)
    mesh = plsc.VectorSubcoreMesh(core_axis_name="c", subcore_axis_name="s")
    return pl.kernel(
        functools.partial(_gather_body, n_chunks=n_chunks),
        out_type=jax.ShapeDtypeStruct((tok, D), jnp.float32),
        mesh=mesh,
        scratch_types=[
            pltpu.VMEM((SUBS, IDX_ROW), jnp.int32),
            pltpu.VMEM((CHUNK, D), jnp.float32),
            pltpu.SemaphoreType.DMA,
        ],
        compiler_params=pltpu.CompilerParams(use_tc_tiling_on_sc=False),
    )


def _dense_body(x_ref, w_ref, b_ref, o_ref):
    acc = jnp.dot(x_ref[...], w_ref[...], preferred_element_type=jnp.float32)
    o_ref[...] = jnp.tanh(acc + b_ref[...])


@functools.lru_cache(maxsize=None)
def _make_dense(tok, bt):
    return pl.pallas_call(
        _dense_body,
        grid=(tok // bt,),
        in_specs=[
            pl.BlockSpec((bt, D), lambda i: (i, 0)),
            pl.BlockSpec((D, F), lambda i: (0, 0)),
            pl.BlockSpec((1, F), lambda i: (0, 0)),
        ],
        out_specs=pl.BlockSpec((bt, F), lambda i: (i, 0)),
        out_shape=jax.ShapeDtypeStruct((tok, F), jnp.float32),
    )


def kernel(inputs, table, W, b):
    B, L = inputs.shape
    tok = B * L
    idx = inputs.reshape(tok // IDX_ROW, IDX_ROW).astype(jnp.int32)
    emb = _make_gather(tok)(idx, table)
    out = _make_dense(tok, 2048)(emb, W, b.reshape(1, F))
    return out.reshape(B, L, F)
